# Initial kernel scaffold; baseline (speedup 1.0000x reference)
import jax
import jax.numpy as jnp
from jax.experimental import pallas as pl


def kernel(x, edge_index, W, att_src, att_dst, bias):
    raise NotImplementedError("write your pallas kernel here")


# speedup vs baseline: 24.9176x; 24.9176x over previous
import dataclasses
import functools

import jax
import jax.numpy as jnp
from jax import lax
from jax.experimental import pallas as pl
from jax.experimental.pallas import tpu as pltpu
from jax.experimental.pallas import tpu_sc as plsc

N = 10000
E = 320000
IN = 128
OUT = 128
H = 2
HC = 32
NPASS = OUT // HC
SW = HC + 16

NSUB = 16
NPAD = 10240
RPS = NPAD // NSUB
BB = 32
EPT = 20032
NB = EPT // BB
EPAD = EPT * NSUB - E
DUMMY = NPAD - 1
WCH = 80


_RB = 2000


def _proj_body(x_ref, w_ref, asv_ref, adv_ref, h_ref, as_ref, ad_ref):
    h = jnp.dot(x_ref[...], w_ref[0], preferred_element_type=jnp.float32)
    for p in range(NPASS):
        h_ref[p, :, :] = h[:, p * HC:(p + 1) * HC]
    as_ref[0, :, 0] = jnp.sum(h * asv_ref[0], axis=1)
    ad_ref[0, :, 0] = jnp.sum(h * adv_ref[0], axis=1)


def _project(x, w_h, att_s, att_d):
    return pl.pallas_call(
        _proj_body,
        grid=(H, N // _RB),
        in_specs=[
            pl.BlockSpec((_RB, IN), lambda c, r: (r, 0)),
            pl.BlockSpec((1, IN, OUT), lambda c, r: (c, 0, 0)),
            pl.BlockSpec((1, 1, OUT), lambda c, r: (c, 0, 0)),
            pl.BlockSpec((1, 1, OUT), lambda c, r: (c, 0, 0)),
        ],
        out_specs=[
            pl.BlockSpec((NPASS, _RB, HC), lambda c, r: (c, r, 0)),
            pl.BlockSpec((1, _RB, 1), lambda c, r: (c, r, 0)),
            pl.BlockSpec((1, _RB, 1), lambda c, r: (c, r, 0)),
        ],
        out_shape=[
            jax.ShapeDtypeStruct((NPASS * H, N, HC), jnp.float32),
            jax.ShapeDtypeStruct((H, N, 1), jnp.float32),
            jax.ShapeDtypeStruct((H, N, 1), jnp.float32),
        ],
    )(x, w_h, att_s, att_d)


def _sc_body(h_hbm, src_hbm, dst_hbm, asrc_hbm, adst_hbm, out_hbm,
             src_v, dst_v, alp_v, asrc_t, adst_t, gin, gout, wb, wc, zb,
             gall, mb, out_sh, gmax_sh, sem_g0, sem_g1, sem_s0, sem_s1):
    cid = lax.axis_index("c")
    sid = lax.axis_index("s")
    f32 = jnp.float32

    pltpu.sync_copy(src_hbm.at[sid], src_v)
    pltpu.sync_copy(dst_hbm.at[sid], dst_v)
    pltpu.sync_copy(asrc_hbm.at[cid], asrc_t)
    pltpu.sync_copy(adst_hbm.at[cid], adst_t)

    zero16 = jnp.zeros((16,), f32)
    lane0 = (lax.iota(jnp.int32, 16) == 0).astype(f32)
    row0 = pl.multiple_of(sid * RPS, 8)

    def zero_my_rows():
        @pl.loop(0, BB)
        def _(r):
            for q in range(SW // 16):
                gout[0, r, pl.ds(q * 16, 16)] = zero16

        @pl.loop(0, RPS, step=BB)
        def _(i):
            pltpu.sync_copy(gout.at[0], out_sh.at[pl.ds(row0 + i, BB)])

    zero_my_rows()

    def alpha_body(j, vmax):
        for q in range(BB // 16):
            qds = pl.ds(q * 16, 16)
            si = src_v[j, qds]
            di = dst_v[j, qds]
            a = plsc.load_gather(asrc_t, [si]) + plsc.load_gather(adst_t, [di])
            a = jnp.where(a > 0, a, 0.2 * a)
            alp_v[j, qds] = a
            vmax = jnp.maximum(vmax, a)
        return vmax

    vmax = lax.fori_loop(0, NB, alpha_body, jnp.full((16,), -1e30, f32))
    mb[...] = vmax
    pltpu.sync_copy(mb, gmax_sh.at[sid])
    plsc.subcore_barrier()
    pltpu.sync_copy(gmax_sh, gall)
    gm = jnp.full((16,), -1e30, f32)
    for t in range(NSUB):
        gm = jnp.maximum(gm, gall[t, :])
    gmax = jnp.max(gm)

    def bump_src(delta):
        @pl.loop(0, NB)
        def _(j):
            for q in range(BB // 16):
                qds = pl.ds(q * 16, 16)
                src_v[j, qds] = src_v[j, qds] + delta

    bump_src(cid * (NPASS * N))

    sem_g = (sem_g0, sem_g1)
    sem_s = (sem_s0, sem_s1)

    def fire_gather(jj, p):
        pltpu.async_copy(h_hbm.at[src_v.at[jj]], gin.at[p], sem_g[p])

    def wait_gather(p):
        pltpu.make_async_copy(h_hbm.at[pl.ds(0, BB)], gin.at[p],
                              sem_g[p]).wait()

    def wait_scatter(p):
        pltpu.make_async_copy(gout.at[p], out_sh.at[pl.ds(0, BB)],
                              sem_s[p]).wait()

    def run_pass(first):
        fire_gather(0, 0)
        fire_gather(1, 1)

        def pipe_body(t, _):
            for p in range(2):
                jj = 2 * t + p
                if first:
                    for q in range(BB // 16):
                        qds = pl.ds(q * 16, 16)
                        alp_v[jj, qds] = jnp.exp(alp_v[jj, qds] - gmax)

                wait_gather(p)

                @pl.when(jj >= 2)
                def _():
                    wait_scatter(p)

                for q2 in range(BB // 16):
                    evec = alp_v[jj, pl.ds(q2 * 16, 16)]
                    for r16 in range(16):
                        r = q2 * 16 + r16
                        ev = evec[r16]
                        for q in range(HC // 16):
                            qds = pl.ds(q * 16, 16)
                            gout[p, r, qds] = gin[p, r, qds] * ev
                        if first:
                            gout[p, r, pl.ds(HC, 16)] = lane0 * ev

                pltpu.async_copy(gout.at[p], out_sh.at[dst_v.at[jj]],
                                 sem_s[p], add=True)

                @pl.when(jj + 2 < NB)
                def _():
                    fire_gather(jj + 2, p)

            return 0

        lax.fori_loop(0, NB // 2, pipe_body, 0)
        wait_scatter(0)
        wait_scatter(1)
        plsc.subcore_barrier()

    def write_out(panel, first):
        for k in range(RPS // WCH):
            @pl.when(sid * RPS + (k + 1) * WCH <= N)
            def _():
                pltpu.sync_copy(out_sh.at[pl.ds(row0 + k * WCH, WCH)], wb)

                if first:
                    @pl.loop(0, WCH, step=16)
                    def _(r0):
                        av = plsc.load_gather(
                            wb, [r0 + lax.iota(jnp.int32, 16),
                                 jnp.full((16,), HC, jnp.int32)])
                        zb[pl.ds(k * WCH + r0, 16)] = 1.0 / (av + 1e-16)

                @pl.loop(0, WCH, step=16)
                def _(r0):
                    ivec = zb[pl.ds(k * WCH + r0, 16)]
                    for r16 in range(16):
                        iv = ivec[r16]
                        for q in range(HC // 16):
                            qds = pl.ds(q * 16, 16)
                            wc[r0 + r16, qds] = wb[r0 + r16, qds] * iv

                dst_row = pl.multiple_of(
                    (cid * NPASS + panel) * N + sid * RPS + k * WCH, 8)
                pltpu.sync_copy(wc, out_hbm.at[pl.ds(dst_row, WCH)])

    run_pass(first=True)
    write_out(0, first=True)

    def panel_body(panel, _):
        zero_my_rows()
        bump_src(N)
        plsc.subcore_barrier()
        run_pass(first=False)
        write_out(panel, first=False)
        return 0

    lax.fori_loop(1, NPASS, panel_body, 0)


def _sc_edge(hq, src2, dst2, asrc_p, adst_p):
    mesh = plsc.VectorSubcoreMesh(core_axis_name="c", subcore_axis_name="s")
    f32 = jnp.float32
    cp = pltpu.CompilerParams(use_tc_tiling_on_sc=False)
    if "needs_layout_passes" in pltpu.CompilerParams.__dataclass_fields__:
        cp = dataclasses.replace(cp, needs_layout_passes=False)
    kern = pl.kernel(
        _sc_body,
        out_type=jax.ShapeDtypeStruct((NPASS * H * N, HC), f32),
        mesh=mesh,
        compiler_params=cp,
        scratch_types=[
            pltpu.VMEM((NB, BB), jnp.int32),
            pltpu.VMEM((NB, BB), jnp.int32),
            pltpu.VMEM((NB, BB), f32),
            pltpu.VMEM((NPAD,), f32),
            pltpu.VMEM((NPAD,), f32),
            pltpu.VMEM((2, BB, HC), f32),
            pltpu.VMEM((2, BB, SW), f32),
            pltpu.VMEM((WCH, SW), f32),
            pltpu.VMEM((WCH, HC), f32),
            pltpu.VMEM((RPS,), f32),
            pltpu.VMEM((NSUB, 16), f32),
            pltpu.VMEM((16,), f32),
            pltpu.VMEM_SHARED((NPAD, SW), f32),
            pltpu.VMEM_SHARED((NSUB, 16), f32),
            pltpu.SemaphoreType.DMA,
            pltpu.SemaphoreType.DMA,
            pltpu.SemaphoreType.DMA,
            pltpu.SemaphoreType.DMA,
        ],
    )
    return kern(hq, src2, dst2, asrc_p, adst_p)


def _comb_body(*refs):
    panels = refs[:2 * NPASS]
    bias_ref = refs[2 * NPASS]
    o_ref = refs[2 * NPASS + 1]
    for p in range(NPASS):
        cols = pl.ds(p * HC, HC)
        o_ref[:, cols] = (0.5 * (panels[p][0] + panels[NPASS + p][0])
                          + bias_ref[0, cols])


def _combine(outq, bias2):
    blk = 2000
    piece = lambda j: pl.BlockSpec((1, blk, HC), lambda i, j=j: (j, i, 0))
    return pl.pallas_call(
        _comb_body,
        grid=(N // blk,),
        in_specs=[piece(j) for j in range(2 * NPASS)]
        + [pl.BlockSpec((1, OUT), lambda i: (0, 0))],
        out_specs=pl.BlockSpec((blk, OUT), lambda i: (i, 0)),
        out_shape=jax.ShapeDtypeStruct((N, OUT), jnp.float32),
    )(*([outq] * (2 * NPASS) + [bias2]))


def kernel(x, edge_index, W, att_src, att_dst, bias):
    w_h = W.reshape(IN, H, OUT).transpose(1, 0, 2)
    att_s = att_src.reshape(H, 1, OUT)
    att_d = att_dst.reshape(H, 1, OUT)

    hq, asrc3, adst3 = _project(x, w_h, att_s, att_d)
    hq = hq.reshape(NPASS * H * N, HC)
    pad = jnp.zeros((H, NPAD - N), jnp.float32)
    asrc_p = jnp.concatenate([asrc3.reshape(H, N), pad], axis=1)
    adst_p = jnp.concatenate([adst3.reshape(H, N), pad], axis=1)

    src = jnp.concatenate(
        [edge_index[0], jnp.zeros((EPAD,), jnp.int32)]).reshape(NSUB, NB, BB)
    dst = jnp.concatenate(
        [edge_index[1], jnp.full((EPAD,), DUMMY, jnp.int32)]
    ).reshape(NSUB, NB, BB)

    outq = _sc_edge(hq, src, dst, asrc_p, adst_p)
    return _combine(outq.reshape(NPASS * H, N, HC), bias.reshape(1, OUT))
